# Initial kernel scaffold; baseline (speedup 1.0000x reference)
import jax
import jax.numpy as jnp
from jax.experimental import pallas as pl


def kernel(x, edge_index, W, b):
    raise NotImplementedError("write your pallas kernel here")


# speedup vs baseline: 9.7570x; 9.7570x over previous
import functools

import jax
import jax.numpy as jnp
from jax import lax
from jax.experimental import pallas as pl
from jax.experimental.pallas import tpu as pltpu
from jax.experimental.pallas import tpu_sc as plsc

NC = 2
NS = 16
NW = NC * NS

CH = 100


def _make_mp(N, D, E):
    epw = E // NW
    nchunk = epw // CH
    rps = (N // NS) // 8 * 8
    tail = N - NS * rps
    mesh = plsc.VectorSubcoreMesh(core_axis_name="c", subcore_axis_name="s")

    @functools.partial(
        pl.kernel,
        mesh=mesh,
        out_type=jax.ShapeDtypeStruct((NC, N, D), jnp.float32),
        scratch_types=[
            pltpu.VMEM((2, 2, CH), jnp.int32),
            pltpu.VMEM((2, CH, D), jnp.float32),
            pltpu.VMEM_SHARED((N, D), jnp.float32),
            pltpu.SemaphoreType.DMA,
            pltpu.SemaphoreType.DMA,
        ],
    )
    def mp(x_hbm, e_hbm, zeros_hbm, out_hbm, idx_v, rows_v, acc, sem_i, sem_r):
        c = lax.axis_index("c")
        s = lax.axis_index("s")
        wid = s * NC + c
        r0 = s * rps
        pltpu.sync_copy(zeros_hbm.at[pl.ds(r0, rps)], acc.at[pl.ds(r0, rps)])

        @pl.when(s == NS - 1)
        def _():
            pltpu.sync_copy(zeros_hbm.at[pl.ds(NS * rps, tail)],
                            acc.at[pl.ds(NS * rps, tail)])

        pltpu.sync_copy(e_hbm.at[wid, 0], idx_v.at[0])
        plsc.subcore_barrier()
        pltpu.async_copy(x_hbm.at[idx_v.at[0, 0]], rows_v.at[0], sem_r)
        pltpu.async_copy(e_hbm.at[wid, 1], idx_v.at[1], sem_i)

        def body(g, _):
            for b in range(2):
                i = g * 2 + b
                nb = 1 - b
                pltpu.make_async_copy(x_hbm.at[idx_v.at[b, 0]], rows_v.at[b],
                                      sem_r).wait()

                @pl.when(i + 1 < nchunk)
                def _():
                    pltpu.make_async_copy(e_hbm.at[wid, i + 1], idx_v.at[nb],
                                          sem_i).wait()
                    pltpu.async_copy(x_hbm.at[idx_v.at[nb, 0]], rows_v.at[nb],
                                     sem_r)

                pltpu.sync_copy(rows_v.at[b], acc.at[idx_v.at[b, 1]], add=True)

                @pl.when(i + 2 < nchunk)
                def _():
                    pltpu.async_copy(e_hbm.at[wid, i + 2], idx_v.at[b], sem_i)
            return _

        lax.fori_loop(0, nchunk // 2, body, None)
        plsc.subcore_barrier()
        pltpu.sync_copy(acc.at[pl.ds(r0, rps)], out_hbm.at[c, pl.ds(r0, rps)])

        @pl.when(s == NS - 1)
        def _():
            pltpu.sync_copy(acc.at[pl.ds(NS * rps, tail)],
                            out_hbm.at[c, pl.ds(NS * rps, tail)])

    return mp


def _linear_body(x_ref, p_ref, w_ref, b_ref, o_ref):
    rst = x_ref[...] + p_ref[0] + p_ref[1]
    o_ref[...] = lax.dot_general(
        rst, w_ref[...], (((1,), (1,)), ((), ())),
        preferred_element_type=jnp.float32) + b_ref[...]


def kernel(x, edge_index, W, b):
    N, D = x.shape
    E = edge_index.shape[1]
    nchunk = (E // NW) // CH
    src = edge_index[0].astype(jnp.int32).reshape(NW, nchunk, 1, CH)
    dst = edge_index[1].astype(jnp.int32).reshape(NW, nchunk, 1, CH)
    edges = jnp.concatenate([src, dst], axis=2)
    zeros = jnp.zeros((N, D), jnp.float32)
    partials = _make_mp(N, D, E)(x, edges, zeros)

    blk = 1000
    grid = N // blk
    out = pl.pallas_call(
        _linear_body,
        grid=(grid,),
        in_specs=[
            pl.BlockSpec((blk, D), lambda i: (i, 0)),
            pl.BlockSpec((NC, blk, D), lambda i: (0, i, 0)),
            pl.BlockSpec((D, D), lambda i: (0, 0)),
            pl.BlockSpec((1, D), lambda i: (0, 0)),
        ],
        out_specs=pl.BlockSpec((blk, D), lambda i: (i, 0)),
        out_shape=jax.ShapeDtypeStruct((N, D), jnp.float32),
    )(x, partials, W, jnp.reshape(b, (1, D)))
    return out
